# baseline (speedup 1.0000x reference)
import jax
import jax.numpy as jnp
from jax.experimental import pallas as pl
from jax.experimental.pallas import tpu as pltpu

_CIN = (1, 32, 32, 64, 64, 128)
_COUT = (32, 32, 64, 64, 128, 128)
_WIN = (64, 32, 16, 8, 4, 2)
_NB = 8


def _band_weights(conv_w, conv_b, cin, cout, wj_lo, wj_hi, wo_lo, wo_hi):
    nwj = wj_hi - wj_lo
    nwo = wo_hi - wo_lo
    cin_pad = conv_w.shape[0] // 9
    w4 = conv_w.reshape(3, 3, cin_pad, -1)[:, :, :cin, :cout].astype(jnp.float32)
    wj = jax.lax.broadcasted_iota(jnp.int32, (nwj, 3, nwo), 0) + wj_lo
    kj = jax.lax.broadcasted_iota(jnp.int32, (nwj, 3, nwo), 1)
    wo = jax.lax.broadcasted_iota(jnp.int32, (nwj, 3, nwo), 2) + wo_lo
    sel = (wj == 2 * wo + kj - 1).astype(jnp.float32)
    band = jnp.einsum('wkv,tkio->twivo', sel, w4)
    band = band.reshape(3 * nwj * cin, nwo * cout).astype(jnp.bfloat16)
    bias = jnp.broadcast_to(conv_b[:1, :cout], (nwo, cout)).reshape(1, nwo * cout)
    return band, bias


_CHUNKS = (
    ((0, 64, 0, 32),),
    ((0, 16, 0, 8), (12, 32, 8, 16)),
    ((0, 8, 0, 4), (4, 16, 4, 8)),
    ((0, 8, 0, 4),),
    ((0, 4, 0, 2),),
    ((0, 2, 0, 1),),
)


def _conv_stack_kernel(x_ref, *refs):
    o_ref = refs[-1]
    nb, hp, wc0 = x_ref.shape
    xc = x_ref[...].reshape(nb * hp, wc0)
    w0, b0 = refs[0], refs[1]
    acc = jnp.dot(xc, w0[...], preferred_element_type=jnp.float32) + b0[...]
    x = jnp.maximum(acc, 0.0).astype(jnp.bfloat16).reshape(nb, hp, w0.shape[1])

    idx = 2
    for li in range(1, 6):
        cin = _CIN[li]
        h, wc = x.shape[1], x.shape[2]
        ho = h // 2
        x2 = x.reshape(nb, ho, 2 * wc)
        even = x2[:, :, :wc]
        odd = x2[:, :, wc:]
        zrow = jnp.zeros((nb, 1, wc), jnp.bfloat16)
        sh = jnp.concatenate([zrow, odd[:, :-1, :]], axis=1)
        outs = []
        for (wj_lo, wj_hi, _, _) in _CHUNKS[li]:
            w_ref, b_ref = refs[idx], refs[idx + 1]
            idx += 2
            lo, hi = wj_lo * cin, wj_hi * cin
            x3 = jnp.concatenate(
                [sh[:, :, lo:hi], even[:, :, lo:hi], odd[:, :, lo:hi]],
                axis=2).reshape(nb * ho, 3 * (hi - lo))
            acc = jnp.dot(x3, w_ref[...], preferred_element_type=jnp.float32)
            outs.append(jnp.maximum(acc + b_ref[...], 0.0))
        x = (outs[0] if len(outs) == 1
             else jnp.concatenate(outs, axis=-1)).astype(jnp.bfloat16)
        x = x.reshape(nb, ho, x.shape[-1])

    o_ref[...] = jnp.transpose(x, (1, 0, 2))


def _gru_head_kernel_fused(x_ref, wih_ref, whh_ref, bih_ref, bhh_ref,
                           wmv_ref, bmv_ref, eps_ref, z_ref, m_ref, v_ref):
    t_len, bb, i_dim = x_ref.shape
    hid = whh_ref.shape[0]
    lat = z_ref.shape[1]
    x = x_ref[...].astype(jnp.float32).reshape(t_len * bb, i_dim)
    gi = jnp.dot(x, wih_ref[...], preferred_element_type=jnp.float32)
    gi = gi + bih_ref[...]
    w_hh = whh_ref[...]
    b_hh = bhh_ref[...]
    h = jnp.zeros((bb, hid), jnp.float32)
    for t in range(t_len):
        g = gi[t * bb:(t + 1) * bb, :]
        gh = jnp.dot(h, w_hh, preferred_element_type=jnp.float32) + b_hh
        r = jax.nn.sigmoid(g[:, :hid] + gh[:, :hid])
        u = jax.nn.sigmoid(g[:, hid:2 * hid] + gh[:, hid:2 * hid])
        c = jnp.tanh(g[:, 2 * hid:] + r * gh[:, 2 * hid:])
        h = c + u * (h - c)
    mv = jnp.dot(h, wmv_ref[...], preferred_element_type=jnp.float32)
    mv = mv + bmv_ref[...]
    mean = mv[:, :lat]
    logvar = mv[:, 128:128 + lat]
    z_ref[...] = eps_ref[...] * jnp.exp(0.5 * logvar) + mean
    m_ref[...] = mean
    v_ref[...] = logvar


def _conv_call(xin, flat):
    B, hp, wc0 = xin.shape
    t_out = hp // 32
    nb = min(_NB, B)
    w_specs = [pl.BlockSpec(a.shape, lambda i: (0, 0)) for a in flat]
    return pl.pallas_call(
        _conv_stack_kernel,
        out_shape=jax.ShapeDtypeStruct((t_out, B, 128), jnp.bfloat16),
        grid=(B // nb,),
        in_specs=[pl.BlockSpec((nb, hp, wc0), lambda i: (i, 0, 0))] + w_specs,
        out_specs=pl.BlockSpec((t_out, nb, 128), lambda i: (0, i, 0)),
        compiler_params=pltpu.CompilerParams(
            dimension_semantics=("parallel",)),
    )(xin, *flat)


def _gru_call(feats, gru_w_ih, gru_w_hh, gru_b_ih, gru_b_hh, head_w, head_b,
              eps):
    t_out, B, _ = feats.shape
    L = eps.shape[1]
    const2 = lambda i: (0, 0)
    return pl.pallas_call(
        _gru_head_kernel_fused,
        out_shape=tuple(jax.ShapeDtypeStruct((B, L), jnp.float32)
                        for _ in range(3)),
        grid=(1,),
        in_specs=[
            pl.BlockSpec((t_out, B, 128), lambda i: (0, i, 0)),
            pl.BlockSpec(gru_w_ih.shape, const2),
            pl.BlockSpec(gru_w_hh.shape, const2),
            pl.BlockSpec(gru_b_ih.shape, const2),
            pl.BlockSpec(gru_b_hh.shape, const2),
            pl.BlockSpec(head_w.shape, const2),
            pl.BlockSpec(head_b.shape, const2),
            pl.BlockSpec((B, L), lambda i: (i, 0)),
        ],
        out_specs=tuple(pl.BlockSpec((B, L), lambda i: (i, 0))
                        for _ in range(3)),
        compiler_params=pltpu.CompilerParams(
            dimension_semantics=("parallel",)),
    )(feats, gru_w_ih, gru_w_hh, gru_b_ih, gru_b_hh, head_w, head_b, eps)


def kernel(conv0_w, conv0_b, conv1_w, conv1_b, conv2_w, conv2_b,
           conv3_w, conv3_b, conv4_w, conv4_b, conv5_w, conv5_b,
           gru_w_ih, gru_w_hh, gru_b_ih, gru_b_hh, head_w, head_b,
           inputs, eps):
    B, T, M = inputs.shape
    conv_params = ((conv0_w, conv0_b), (conv1_w, conv1_b), (conv2_w, conv2_b),
                   (conv3_w, conv3_b), (conv4_w, conv4_b), (conv5_w, conv5_b))
    flat = []
    for i, (cw, cb) in enumerate(conv_params):
        for (wj_lo, wj_hi, wo_lo, wo_hi) in _CHUNKS[i]:
            band, bias = _band_weights(cw, cb, _CIN[i], _COUT[i],
                                       wj_lo, wj_hi, wo_lo, wo_hi)
            flat.extend((band, bias))
    flat = tuple(flat)

    x2 = inputs.astype(jnp.bfloat16).reshape(B, T // 2, 2 * M)
    sh_odd = jnp.pad(x2[:, :-1, M:], ((0, 0), (1, 0), (0, 0)))
    xin = jnp.concatenate([sh_odd, x2], axis=2)
    gru_args = (gru_w_ih, gru_w_hh, gru_b_ih, gru_b_hh, head_w, head_b)

    feats = _conv_call(xin, flat)
    z, mean, logvar = _gru_call(feats, *gru_args, eps)
    return z[:, None, :], (mean, logvar)


# speedup vs baseline: 1.0866x; 1.0866x over previous
import jax
import jax.numpy as jnp
from jax.experimental import pallas as pl
from jax.experimental.pallas import tpu as pltpu

_CIN = (1, 32, 32, 64, 64, 128)
_COUT = (32, 32, 64, 64, 128, 128)
_WIN = (64, 32, 16, 8, 4, 2)
_NB = 8


def _band_weights(conv_w, conv_b, cin, cout, wj_lo, wj_hi, wo_lo, wo_hi):
    nwj = wj_hi - wj_lo
    nwo = wo_hi - wo_lo
    cin_pad = conv_w.shape[0] // 9
    w4 = conv_w.reshape(3, 3, cin_pad, -1)[:, :, :cin, :cout].astype(jnp.float32)
    wj = jax.lax.broadcasted_iota(jnp.int32, (nwj, 3, nwo), 0) + wj_lo
    kj = jax.lax.broadcasted_iota(jnp.int32, (nwj, 3, nwo), 1)
    wo = jax.lax.broadcasted_iota(jnp.int32, (nwj, 3, nwo), 2) + wo_lo
    sel = (wj == 2 * wo + kj - 1).astype(jnp.float32)
    band = jnp.einsum('wkv,tkio->twivo', sel, w4)
    band = band.reshape(3 * nwj * cin, nwo * cout).astype(jnp.bfloat16)
    bias = jnp.broadcast_to(conv_b[:1, :cout], (nwo, cout)).reshape(1, nwo * cout)
    return band, bias


_CHUNKS = (
    ((0, 64, 0, 32),),
    ((0, 16, 0, 8), (12, 32, 8, 16)),
    ((0, 8, 0, 4), (4, 16, 4, 8)),
    ((0, 8, 0, 4),),
    ((0, 4, 0, 2),),
    ((0, 2, 0, 1),),
)


def _conv_stack_kernel(x_ref, *refs):
    o_ref = refs[-1]
    nb, hp, wc0 = x_ref.shape
    x2 = x_ref[...].astype(jnp.bfloat16)
    zrow = jnp.zeros((nb, 1, wc0), jnp.bfloat16)
    sh = jnp.concatenate([zrow, x2[:, :-1, :]], axis=1)
    xc = jnp.concatenate([sh, x2], axis=2).reshape(nb * hp, 2 * wc0)
    w0, b0 = refs[0], refs[1]
    acc = jnp.dot(xc, w0[...], preferred_element_type=jnp.float32) + b0[...]
    x = jnp.maximum(acc, 0.0).astype(jnp.bfloat16).reshape(nb, hp, w0.shape[1])

    idx = 2
    for li in range(1, 6):
        cin = _CIN[li]
        h, wc = x.shape[1], x.shape[2]
        ho = h // 2
        x2 = x.reshape(nb, ho, 2 * wc)
        even = x2[:, :, :wc]
        odd = x2[:, :, wc:]
        zrow = jnp.zeros((nb, 1, wc), jnp.bfloat16)
        sh = jnp.concatenate([zrow, odd[:, :-1, :]], axis=1)
        outs = []
        for (wj_lo, wj_hi, _, _) in _CHUNKS[li]:
            w_ref, b_ref = refs[idx], refs[idx + 1]
            idx += 2
            lo, hi = wj_lo * cin, wj_hi * cin
            x3 = jnp.concatenate(
                [sh[:, :, lo:hi], even[:, :, lo:hi], odd[:, :, lo:hi]],
                axis=2).reshape(nb * ho, 3 * (hi - lo))
            acc = jnp.dot(x3, w_ref[...], preferred_element_type=jnp.float32)
            outs.append(jnp.maximum(acc + b_ref[...], 0.0))
        x = (outs[0] if len(outs) == 1
             else jnp.concatenate(outs, axis=-1)).astype(jnp.bfloat16)
        x = x.reshape(nb, ho, x.shape[-1])

    o_ref[...] = jnp.transpose(x, (1, 0, 2))


def _gru_head_kernel_fused(x_ref, wih_ref, whh_ref, bih_ref, bhh_ref,
                           wmv_ref, bmv_ref, eps_ref, z_ref, m_ref, v_ref):
    t_len, bb, i_dim = x_ref.shape
    hid = whh_ref.shape[0]
    lat = z_ref.shape[1]
    x = x_ref[...].astype(jnp.float32).reshape(t_len * bb, i_dim)
    gi = jnp.dot(x, wih_ref[...], preferred_element_type=jnp.float32)
    gi = gi + bih_ref[...]
    w_hh = whh_ref[...]
    b_hh = bhh_ref[...]
    h = jnp.zeros((bb, hid), jnp.float32)
    for t in range(t_len):
        g = gi[t * bb:(t + 1) * bb, :]
        gh = jnp.dot(h, w_hh, preferred_element_type=jnp.float32) + b_hh
        r = jax.nn.sigmoid(g[:, :hid] + gh[:, :hid])
        u = jax.nn.sigmoid(g[:, hid:2 * hid] + gh[:, hid:2 * hid])
        c = jnp.tanh(g[:, 2 * hid:] + r * gh[:, 2 * hid:])
        h = c + u * (h - c)
    mv = jnp.dot(h, wmv_ref[...], preferred_element_type=jnp.float32)
    mv = mv + bmv_ref[...]
    mean = mv[:, :lat]
    logvar = mv[:, 128:128 + lat]
    z_ref[...] = eps_ref[...] * jnp.exp(0.5 * logvar) + mean
    m_ref[...] = mean
    v_ref[...] = logvar


def _conv_call(xin, flat):
    B, hp, wc0 = xin.shape
    t_out = hp // 32
    nb = min(_NB, B)
    w_specs = [pl.BlockSpec(a.shape, lambda i: (0, 0)) for a in flat]
    return pl.pallas_call(
        _conv_stack_kernel,
        out_shape=jax.ShapeDtypeStruct((t_out, B, 128), jnp.bfloat16),
        grid=(B // nb,),
        in_specs=[pl.BlockSpec((nb, hp, wc0), lambda i: (i, 0, 0))] + w_specs,
        out_specs=pl.BlockSpec((t_out, nb, 128), lambda i: (0, i, 0)),
        compiler_params=pltpu.CompilerParams(
            dimension_semantics=("parallel",)),
    )(xin, *flat)


def _gru_call(feats, gru_w_ih, gru_w_hh, gru_b_ih, gru_b_hh, head_w, head_b,
              eps):
    t_out, B, _ = feats.shape
    L = eps.shape[1]
    const2 = lambda i: (0, 0)
    return pl.pallas_call(
        _gru_head_kernel_fused,
        out_shape=tuple(jax.ShapeDtypeStruct((B, L), jnp.float32)
                        for _ in range(3)),
        grid=(1,),
        in_specs=[
            pl.BlockSpec((t_out, B, 128), lambda i: (0, i, 0)),
            pl.BlockSpec(gru_w_ih.shape, const2),
            pl.BlockSpec(gru_w_hh.shape, const2),
            pl.BlockSpec(gru_b_ih.shape, const2),
            pl.BlockSpec(gru_b_hh.shape, const2),
            pl.BlockSpec(head_w.shape, const2),
            pl.BlockSpec(head_b.shape, const2),
            pl.BlockSpec((B, L), lambda i: (i, 0)),
        ],
        out_specs=tuple(pl.BlockSpec((B, L), lambda i: (i, 0))
                        for _ in range(3)),
        compiler_params=pltpu.CompilerParams(
            dimension_semantics=("parallel",)),
    )(feats, gru_w_ih, gru_w_hh, gru_b_ih, gru_b_hh, head_w, head_b, eps)


def kernel(conv0_w, conv0_b, conv1_w, conv1_b, conv2_w, conv2_b,
           conv3_w, conv3_b, conv4_w, conv4_b, conv5_w, conv5_b,
           gru_w_ih, gru_w_hh, gru_b_ih, gru_b_hh, head_w, head_b,
           inputs, eps):
    B, T, M = inputs.shape
    conv_params = ((conv0_w, conv0_b), (conv1_w, conv1_b), (conv2_w, conv2_b),
                   (conv3_w, conv3_b), (conv4_w, conv4_b), (conv5_w, conv5_b))
    flat = []
    for i, (cw, cb) in enumerate(conv_params):
        for (wj_lo, wj_hi, wo_lo, wo_hi) in _CHUNKS[i]:
            band, bias = _band_weights(cw, cb, _CIN[i], _COUT[i],
                                       wj_lo, wj_hi, wo_lo, wo_hi)
            if i == 0:
                g = _WIN[0] * _CIN[0]
                band = jnp.concatenate(
                    [jnp.zeros((g, band.shape[1]), band.dtype), band])
            flat.extend((band, bias))
    flat = tuple(flat)

    xin = inputs.reshape(B, T // 2, 2 * M)
    gru_args = (gru_w_ih, gru_w_hh, gru_b_ih, gru_b_hh, head_w, head_b)

    feats = _conv_call(xin, flat)
    z, mean, logvar = _gru_call(feats, *gru_args, eps)
    return z[:, None, :], (mean, logvar)
